# baseline (speedup 1.0000x reference)
import functools

import jax
import jax.numpy as jnp
from jax.experimental import pallas as pl
from jax.experimental.pallas import tpu as pltpu

_EC = 4


def _attn_kernel(wa_ref, sa_ref, ww_ref, sw_ref, mask_ref, w1_ref, b1_ref,
                 w2_ref, w3_ref, b3_ref, out_ref, *, E):
    wa = wa_ref[0]
    sa = sa_ref[0]
    ww = ww_ref[0]
    sw = sw_ref[0]
    mrow = mask_ref[0]
    w1 = w1_ref[...]
    b1 = b1_ref[...]
    w2 = w2_ref[...]
    w3 = w3_ref[...]
    b3 = b3_ref[...]
    wd = wa.shape[1]

    w1b = w1.astype(jnp.bfloat16)
    p = (jnp.dot(wa.astype(jnp.bfloat16), w1b[:wd],
                 preferred_element_type=jnp.float32)
         + jnp.dot(sa.astype(jnp.bfloat16), w1b[wd:],
                   preferred_element_type=jnp.float32))
    q = (jnp.dot(ww, w1[:wd], preferred_element_type=jnp.float32)
         + jnp.dot(sw, w1[wd:], preferred_element_type=jnp.float32)
         - b1)

    L, mid = p.shape
    w2row = w2.reshape(1, 1, mid)
    rows = []
    for e0 in range(0, E, _EC):
        qc = q[e0:e0 + _EC]
        t = jnp.tanh(p[None, :, :] - qc[:, None, :])
        rows.append(jnp.sum(t * w2row, axis=-1))
    s = jnp.concatenate(rows, axis=0)

    s = jnp.where(mrow > 0.0, s, -1e30)
    s = s - jnp.max(s, axis=-1, keepdims=True)
    ex = jnp.exp(s)
    att = ex / jnp.sum(ex, axis=-1, keepdims=True)

    pooled_w = jnp.dot(att, wa, preferred_element_type=jnp.float32)
    pooled_s = jnp.dot(att, sa, preferred_element_type=jnp.float32)
    out = (jnp.dot(pooled_w + ww, w3[:wd], preferred_element_type=jnp.float32)
           + jnp.dot(pooled_s + sw, w3[wd:], preferred_element_type=jnp.float32)
           + b3)
    out_ref[0] = out


@jax.jit
def kernel(word_all, word_weighted, syn_all, syn_weighted, word_mask,
           W1, b1, W2, b2, W3, b3):
    b, e, w, wd = word_all.shape
    sd = syn_all.shape[-1]
    h = wd + sd
    L = e * w
    mid = W1.shape[-1]

    wa = word_all.reshape(b, L, wd)
    sa = syn_all.reshape(b, L, sd)
    mask = word_mask.reshape(b, 1, L)

    b1r = b1.reshape(1, mid)
    w2r = W2.reshape(1, mid)
    b3r = b3.reshape(1, mid)

    grid = (b,)
    out = pl.pallas_call(
        functools.partial(_attn_kernel, E=e),
        grid=grid,
        in_specs=[
            pl.BlockSpec((1, L, wd), lambda i: (i, 0, 0)),
            pl.BlockSpec((1, L, sd), lambda i: (i, 0, 0)),
            pl.BlockSpec((1, e, wd), lambda i: (i, 0, 0)),
            pl.BlockSpec((1, e, sd), lambda i: (i, 0, 0)),
            pl.BlockSpec((1, 1, L), lambda i: (i, 0, 0)),
            pl.BlockSpec((h, mid), lambda i: (0, 0)),
            pl.BlockSpec((1, mid), lambda i: (0, 0)),
            pl.BlockSpec((1, mid), lambda i: (0, 0)),
            pl.BlockSpec((h, mid), lambda i: (0, 0)),
            pl.BlockSpec((1, mid), lambda i: (0, 0)),
        ],
        out_specs=pl.BlockSpec((1, e, mid), lambda i: (i, 0, 0)),
        out_shape=jax.ShapeDtypeStruct((b, e, mid), jnp.float32),
        compiler_params=pltpu.CompilerParams(
            dimension_semantics=("parallel",),
        ),
    )(wa, sa, word_weighted, syn_weighted, mask, W1, b1r, w2r, W3, b3r)
    return out


# speedup vs baseline: 1.0071x; 1.0071x over previous
import functools

import jax
import jax.numpy as jnp
from jax.experimental import pallas as pl
from jax.experimental.pallas import tpu as pltpu

_EC = 4


def _attn_kernel(wa_ref, sa_ref, ww_ref, sw_ref, mask_ref, w1_ref, b1_ref,
                 w2_ref, w3_ref, b3_ref, out_ref, *, E):
    wa = wa_ref[0]
    sa = sa_ref[0]
    ww = ww_ref[0]
    sw = sw_ref[0]
    mrow = mask_ref[0]
    w1 = w1_ref[...]
    b1 = b1_ref[...]
    w2 = w2_ref[...]
    w3 = w3_ref[...]
    b3 = b3_ref[...]

    emb = jnp.concatenate([wa, sa], axis=-1)
    wtd = jnp.concatenate([ww, sw], axis=-1)
    p = jnp.dot(emb, w1, preferred_element_type=jnp.float32)
    q = jnp.dot(wtd, w1, preferred_element_type=jnp.float32) - b1

    L, mid = p.shape
    w2row = w2.reshape(1, 1, mid)
    rows = []
    for e0 in range(0, E, _EC):
        qc = q[e0:e0 + _EC]
        t = jnp.tanh(p[None, :, :] - qc[:, None, :])
        rows.append(jnp.sum(t * w2row, axis=-1))
    s = jnp.concatenate(rows, axis=0)

    s = jnp.where(mrow > 0.0, s, -1e30)
    s = s - jnp.max(s, axis=-1, keepdims=True)
    ex = jnp.exp(s)
    att = ex / jnp.sum(ex, axis=-1, keepdims=True)

    pooled = jnp.dot(att, emb, preferred_element_type=jnp.float32)
    out = jnp.dot(pooled + wtd, w3,
                  preferred_element_type=jnp.float32) + b3
    out_ref[0] = out


@jax.jit
def kernel(word_all, word_weighted, syn_all, syn_weighted, word_mask,
           W1, b1, W2, b2, W3, b3):
    b, e, w, wd = word_all.shape
    sd = syn_all.shape[-1]
    h = wd + sd
    L = e * w
    mid = W1.shape[-1]

    wa = word_all.reshape(b, L, wd)
    sa = syn_all.reshape(b, L, sd)
    mask = word_mask.reshape(b, 1, L)

    b1r = b1.reshape(1, mid)
    w2r = W2.reshape(1, mid)
    b3r = b3.reshape(1, mid)

    grid = (b,)
    out = pl.pallas_call(
        functools.partial(_attn_kernel, E=e),
        grid=grid,
        in_specs=[
            pl.BlockSpec((1, L, wd), lambda i: (i, 0, 0)),
            pl.BlockSpec((1, L, sd), lambda i: (i, 0, 0)),
            pl.BlockSpec((1, e, wd), lambda i: (i, 0, 0)),
            pl.BlockSpec((1, e, sd), lambda i: (i, 0, 0)),
            pl.BlockSpec((1, 1, L), lambda i: (i, 0, 0)),
            pl.BlockSpec((h, mid), lambda i: (0, 0)),
            pl.BlockSpec((1, mid), lambda i: (0, 0)),
            pl.BlockSpec((1, mid), lambda i: (0, 0)),
            pl.BlockSpec((h, mid), lambda i: (0, 0)),
            pl.BlockSpec((1, mid), lambda i: (0, 0)),
        ],
        out_specs=pl.BlockSpec((1, e, mid), lambda i: (i, 0, 0)),
        out_shape=jax.ShapeDtypeStruct((b, e, mid), jnp.float32),
        compiler_params=pltpu.CompilerParams(
            dimension_semantics=("parallel",),
        ),
    )(wa, sa, word_weighted, syn_weighted, mask, W1, b1r, w2r, W3, b3r)
    return out


# speedup vs baseline: 1.0131x; 1.0060x over previous
import functools

import jax
import jax.numpy as jnp
from jax.experimental import pallas as pl
from jax.experimental.pallas import tpu as pltpu

_EC = 4


def _attn_kernel(wa_ref, sa_ref, ww_ref, sw_ref, mask_ref, w1_ref, b1_ref,
                 w2_ref, w3_ref, b3_ref, out_ref, *, E):
    wa = wa_ref[0]
    sa = sa_ref[0]
    ww = ww_ref[0]
    sw = sw_ref[0]
    mrow = mask_ref[0]
    w1 = w1_ref[...]
    b1 = b1_ref[...]
    w2 = w2_ref[...]
    w3 = w3_ref[...]
    b3 = b3_ref[...]

    emb = jnp.concatenate([wa, sa], axis=-1)
    wtd = jnp.concatenate([ww, sw], axis=-1)
    p = jnp.dot(emb, w1, preferred_element_type=jnp.float32)
    q = jnp.dot(wtd, w1, preferred_element_type=jnp.float32) - b1

    L, mid = p.shape
    w2row = w2.reshape(1, 1, mid)
    rows = []
    for e0 in range(0, E, _EC):
        qc = q[e0:e0 + _EC]
        t = jnp.tanh(p[None, :, :] - qc[:, None, :])
        rows.append(jnp.sum(t * w2row, axis=-1))
    s = jnp.concatenate(rows, axis=0)

    s = jnp.where(mrow > 0.0, s, -1e30)
    s = s - jnp.max(s, axis=-1, keepdims=True)
    ex = jnp.exp(s)
    att = ex / jnp.sum(ex, axis=-1, keepdims=True)

    pooled = jnp.dot(att, emb, preferred_element_type=jnp.float32)
    out = jnp.dot(pooled + wtd, w3,
                  preferred_element_type=jnp.float32) + b3
    out_ref[0] = out


@jax.jit
def kernel(word_all, word_weighted, syn_all, syn_weighted, word_mask,
           W1, b1, W2, b2, W3, b3):
    b, e, w, wd = word_all.shape
    sd = syn_all.shape[-1]
    h = wd + sd
    L = e * w
    mid = W1.shape[-1]

    wa = word_all.reshape(b, L, wd)
    sa = syn_all.reshape(b, L, sd)
    mask = word_mask.reshape(b, 1, L)

    b1r = b1.reshape(1, mid)
    w2r = W2.reshape(1, mid)
    b3r = b3.reshape(1, mid)

    grid = (b,)
    out = pl.pallas_call(
        functools.partial(_attn_kernel, E=e),
        grid=grid,
        in_specs=[
            pl.BlockSpec((1, L, wd), lambda i: (i, 0, 0)),
            pl.BlockSpec((1, L, sd), lambda i: (i, 0, 0)),
            pl.BlockSpec((1, e, wd), lambda i: (i, 0, 0)),
            pl.BlockSpec((1, e, sd), lambda i: (i, 0, 0)),
            pl.BlockSpec((1, 1, L), lambda i: (i, 0, 0)),
            pl.BlockSpec((h, mid), lambda i: (0, 0)),
            pl.BlockSpec((1, mid), lambda i: (0, 0)),
            pl.BlockSpec((1, mid), lambda i: (0, 0)),
            pl.BlockSpec((h, mid), lambda i: (0, 0)),
            pl.BlockSpec((1, mid), lambda i: (0, 0)),
        ],
        out_specs=pl.BlockSpec((1, e, mid), lambda i: (i, 0, 0)),
        out_shape=jax.ShapeDtypeStruct((b, e, mid), jnp.float32),
        compiler_params=pltpu.CompilerParams(
            dimension_semantics=("arbitrary",),
        ),
    )(wa, sa, word_weighted, syn_weighted, mask, W1, b1r, w2r, W3, b3r)
    return out
